# baseline (speedup 1.0000x reference)
import functools

import jax
import jax.numpy as jnp
from jax import lax
from jax.experimental import pallas as pl
from jax.experimental.pallas import tpu as pltpu
from jax.experimental.pallas import tpu_sc as plsc

N = 10000
E = 320000
D = 128
DIN = 128

NC = 2
NS = 16
NW = NC * NS
CPT = D // NW
L = 16

def _mesh():
    return plsc.VectorSubcoreMesh(core_axis_name="c", subcore_axis_name="s",
                                  num_cores=NC, num_subcores=NS)


_SC_PARAMS = pltpu.CompilerParams(needs_layout_passes=False,
                                  use_tc_tiling_on_sc=False)

CHD = 2000
CHM = 3200
NCHM = E // CHM
EPW = E // NW


def _wid():
    return lax.axis_index("s") * NC + lax.axis_index("c")


def _deg_body(eif_hbm, ew_hbm, perm_hbm, degp_hbm, ndst_hbm,
              deg_v, di_v, ew_v, pm_v, nd_v, sem):
    w = _wid()
    zero = jnp.zeros((L,), jnp.float32)

    @plsc.parallel_loop(0, N // L, 1, unroll=8)
    def zbody(i):
        deg_v[pl.ds(i * L, L)] = zero

    base = w * EPW

    def cbody(c, _):
        off = base + c * CHD
        pltpu.sync_copy(eif_hbm.at[pl.ds(E + off, CHD)], di_v)
        pltpu.sync_copy(ew_hbm.at[pl.ds(off, CHD)], ew_v)
        pltpu.sync_copy(perm_hbm.at[pl.ds(off, CHD)], pm_v)
        eoff = jnp.full((L,), E, jnp.int32)

        @plsc.parallel_loop(0, CHD // L, 1, unroll=8)
        def pbody(i):
            sl = pl.ds(i * L, L)
            pm_v[sl] = pm_v[sl] + eoff

        cp = pltpu.async_copy(eif_hbm.at[pm_v], nd_v, sem)

        @plsc.parallel_loop(0, CHD // L, 1, unroll=8)
        def gbody(g):
            idx = di_v[pl.ds(g * L, L)]
            val = ew_v[pl.ds(g * L, L)]
            plsc.addupdate_scatter(deg_v, [idx], val)

        cp.wait()
        pltpu.sync_copy(nd_v, ndst_hbm.at[pl.ds(off, CHD)])
        return 0

    lax.fori_loop(0, EPW // CHD, cbody, 0)
    pltpu.sync_copy(deg_v, degp_hbm.at[w])


BR = 1024
GR = (N + BR - 1) // BR


def _pack_cols(x):
    lo = lax.bitcast_convert_type(x[:, :D // 2].astype(jnp.bfloat16),
                                  jnp.uint16).astype(jnp.uint32)
    hi = lax.bitcast_convert_type(x[:, D // 2:].astype(jnp.bfloat16),
                                  jnp.uint16).astype(jnp.uint32)
    return lax.bitcast_convert_type(lo | (hi << 16), jnp.int32)


def _prep_body(x_ref, pw_ref, pb_ref, emb_ref, degp_ref, ht_ref, hp_ref, dinv_ref):
    hb = (
        jnp.dot(x_ref[...], pw_ref[...], preferred_element_type=jnp.float32)
        + pb_ref[...]
        + emb_ref[...]
    )
    ht_ref[...] = hb.T
    deg = jnp.sum(degp_ref[...], axis=0, keepdims=True) + 1.0
    dv = jnp.where(deg > 0, lax.rsqrt(deg), 0.0)
    dinv_ref[...] = dv
    hp_ref[...] = _pack_cols(hb * dv.T).T


def _prep(x, pw, pb, emb, degp):
    return pl.pallas_call(
        _prep_body,
        grid=(GR,),
        in_specs=[
            pl.BlockSpec((BR, DIN), lambda i: (i, 0)),
            pl.BlockSpec((DIN, D), lambda i: (0, 0)),
            pl.BlockSpec((1, D), lambda i: (0, 0)),
            pl.BlockSpec((BR, D), lambda i: (i, 0)),
            pl.BlockSpec((NW, BR), lambda i: (0, i)),
        ],
        out_specs=[
            pl.BlockSpec((D, BR), lambda i: (0, i)),
            pl.BlockSpec((D // 2, BR), lambda i: (0, i)),
            pl.BlockSpec((1, BR), lambda i: (0, i)),
        ],
        out_shape=[
            jax.ShapeDtypeStruct((D, N), jnp.float32),
            jax.ShapeDtypeStruct((D // 2, N), jnp.int32),
            jax.ShapeDtypeStruct((1, N), jnp.float32),
        ],
    )(x, pw, pb, emb, degp)


def _main_body(hp_hbm, dinv_hbm, ei_hbm, ew_hbm, q_hbm,
               dinv_v, h_v, q_v,
               src0, src1, dst0, dst1, ew0, ew1, se0, se1):
    w = _wid()
    pltpu.sync_copy(dinv_hbm.at[0], dinv_v)
    pltpu.sync_copy(hp_hbm.at[pl.ds(2 * w, 2)], h_v)

    zero = jnp.zeros((L,), jnp.float32)
    for r in range(CPT):
        @plsc.parallel_loop(0, N // L, 1, unroll=8)
        def zbody(i, r=r):
            q_v[r, pl.ds(i * L, L)] = zero

    bufs = ((src0, dst0, ew0, se0), (src1, dst1, ew1, se1))

    def issue(c, b):
        sv, dv, wv, sem = bufs[b]
        off = c * CHM
        pltpu.async_copy(ei_hbm.at[0, pl.ds(off, CHM)], sv, sem)
        pltpu.async_copy(ei_hbm.at[1, pl.ds(off, CHM)], dv, sem)
        pltpu.async_copy(ew_hbm.at[pl.ds(off, CHM)], wv, sem)

    def wait(b):
        sv, dv, wv, sem = bufs[b]
        pltpu.make_async_copy(ei_hbm.at[0, pl.ds(0, CHM)], sv, sem).wait()
        pltpu.make_async_copy(ei_hbm.at[1, pl.ds(0, CHM)], dv, sem).wait()
        pltpu.make_async_copy(ew_hbm.at[pl.ds(0, CHM)], wv, sem).wait()

    def compute(b):
        sv, dv, wv, _ = bufs[b]

        @plsc.parallel_loop(0, CHM // L, 1, unroll=8)
        def gbody(g):
            s16 = sv[pl.ds(g * L, L)]
            d16 = dv[pl.ds(g * L, L)]
            w16 = wv[pl.ds(g * L, L)]
            for r2 in range(2):
                rf = jnp.full((L,), r2, jnp.int32)
                pg = plsc.load_gather(h_v, [rf, s16])
                lo, hi = plsc.unpack(plsc.bitcast(pg, jnp.bfloat16),
                                     format=plsc.PackFormat.INTERLEAVED)
                plsc.addupdate_scatter(q_v, [rf, d16], lo * w16)
                rf2 = jnp.full((L,), 2 + r2, jnp.int32)
                plsc.addupdate_scatter(q_v, [rf2, d16], hi * w16)

    issue(0, 0)

    def pair(c2, _):
        c = 2 * c2
        wait(0)
        issue(c + 1, 1)
        compute(0)
        wait(1)

        @pl.when(c2 < NCHM // 2 - 1)
        def _issue_next():
            issue(c + 2, 0)

        compute(1)
        return 0

    lax.fori_loop(0, NCHM // 2, pair, 0)

    for r in range(CPT):
        @plsc.parallel_loop(0, N // L, 1, unroll=4)
        def qbody(i, r=r):
            sl = pl.ds(i * L, L)
            q_v[r, sl] = q_v[r, sl] * dinv_v[sl]

    pltpu.sync_copy(q_v.at[pl.ds(0, 2)], q_hbm.at[pl.ds(2 * w, 2)])
    pltpu.sync_copy(q_v.at[pl.ds(2, 2)], q_hbm.at[pl.ds(D // 2 + 2 * w, 2)])


def _z_body(q_ref, dinv_ref, ht_ref, czw_ref, czb_ref, lzw_ref, lzb_ref,
            chw_ref, chb_ref, lhw_ref, lhb_ref, z_ref, zp_ref):
    wz = jnp.dot(czw_ref[...], lzw_ref[...], preferred_element_type=jnp.float32)
    bz = jnp.dot(czb_ref[...], lzw_ref[...], preferred_element_type=jnp.float32) + lzb_ref[...]
    wh = jnp.dot(chw_ref[...], lhw_ref[...], preferred_element_type=jnp.float32)
    bh = jnp.dot(chb_ref[...], lhw_ref[...], preferred_element_type=jnp.float32) + lhb_ref[...]
    dv = dinv_ref[...]
    pt = q_ref[...] + dv * dv * ht_ref[...]
    mz = lax.dot_general(pt, wz, (((0,), (0,)), ((), ())),
                         preferred_element_type=jnp.float32)
    mh = lax.dot_general(pt, wh, (((0,), (0,)), ((), ())),
                         preferred_element_type=jnp.float32)
    zb = (1.0 - jax.nn.sigmoid(mz + bz)) * jnp.tanh(mh + bh)
    z_ref[...] = zb
    zp_ref[...] = _pack_cols(zb).T


def _zk(q, dinv, ht, czw, czb, lzw, lzb, chw, chb, lhw, lhb):
    full = lambda i: (0, 0)
    return pl.pallas_call(
        _z_body,
        grid=(GR,),
        in_specs=[
            pl.BlockSpec((D, BR), lambda i: (0, i)),
            pl.BlockSpec((1, BR), lambda i: (0, i)),
            pl.BlockSpec((D, BR), lambda i: (0, i)),
            pl.BlockSpec((D, D), full),
            pl.BlockSpec((1, D), full),
            pl.BlockSpec((D, D), full),
            pl.BlockSpec((1, D), full),
            pl.BlockSpec((D, D), full),
            pl.BlockSpec((1, D), full),
            pl.BlockSpec((D, D), full),
            pl.BlockSpec((1, D), full),
        ],
        out_specs=[
            pl.BlockSpec((BR, D), lambda i: (i, 0)),
            pl.BlockSpec((D // 2, BR), lambda i: (0, i)),
        ],
        out_shape=[
            jax.ShapeDtypeStruct((N, D), jnp.float32),
            jax.ShapeDtypeStruct((D // 2, N), jnp.int32),
        ],
    )(q, dinv, ht, czw, czb, lzw, lzb, chw, chb, lhw, lhb)


E2 = E // 2
NCH2 = E2 // CHM


def _dec_body(zt_hbm, ei_hbm, ndst_hbm, pos_hbm, neg_hbm,
              z_v, src0, src1, dst0, dst1, nd0, nd1,
              pos0, pos1, neg0, neg1, se0, se1, so0, so1):
    w = _wid()
    g = w // 2
    half = w % 2
    ebase = half * E2
    pltpu.sync_copy(zt_hbm.at[pl.ds(4 * g, 4)], z_v)
    ebufs = ((src0, dst0, nd0, se0), (src1, dst1, nd1, se1))
    obufs = ((pos0, neg0, so0), (pos1, neg1, so1))

    def issue(c, b):
        sv, dv, nv, sem = ebufs[b]
        off = ebase + c * CHM
        pltpu.async_copy(ei_hbm.at[0, pl.ds(off, CHM)], sv, sem)
        pltpu.async_copy(ei_hbm.at[1, pl.ds(off, CHM)], dv, sem)
        pltpu.async_copy(ndst_hbm.at[pl.ds(off, CHM)], nv, sem)

    def wait(b):
        sv, dv, nv, sem = ebufs[b]
        pltpu.make_async_copy(ei_hbm.at[0, pl.ds(0, CHM)], sv, sem).wait()
        pltpu.make_async_copy(ei_hbm.at[1, pl.ds(0, CHM)], dv, sem).wait()
        pltpu.make_async_copy(ndst_hbm.at[pl.ds(0, CHM)], nv, sem).wait()

    def wait_out(b):
        pv, ngv, sem = obufs[b]
        pltpu.make_async_copy(pv, pos_hbm.at[g, pl.ds(0, CHM)], sem).wait()
        pltpu.make_async_copy(ngv, neg_hbm.at[g, pl.ds(0, CHM)], sem).wait()

    def compute(c, b):
        sv, dv, nv, _ = ebufs[b]
        pv, ngv, osem = obufs[b]

        @plsc.parallel_loop(0, CHM // L, 1, unroll=8)
        def gbody(g):
            s16 = sv[pl.ds(g * L, L)]
            d16 = dv[pl.ds(g * L, L)]
            n16 = nv[pl.ds(g * L, L)]
            pacc = jnp.zeros((L,), jnp.float32)
            nacc = jnp.zeros((L,), jnp.float32)
            for r2 in range(4):
                rf = jnp.full((L,), r2, jnp.int32)
                fmt = plsc.PackFormat.INTERLEAVED
                zsl, zsh = plsc.unpack(
                    plsc.bitcast(plsc.load_gather(z_v, [rf, s16]),
                                 jnp.bfloat16), format=fmt)
                zdl, zdh = plsc.unpack(
                    plsc.bitcast(plsc.load_gather(z_v, [rf, d16]),
                                 jnp.bfloat16), format=fmt)
                znl, znh = plsc.unpack(
                    plsc.bitcast(plsc.load_gather(z_v, [rf, n16]),
                                 jnp.bfloat16), format=fmt)
                pacc = pacc + zsl * zdl + zsh * zdh
                nacc = nacc + zsl * znl + zsh * znh
            pv[pl.ds(g * L, L)] = pacc
            ngv[pl.ds(g * L, L)] = nacc

        off = ebase + c * CHM
        pltpu.async_copy(pv, pos_hbm.at[g, pl.ds(off, CHM)], osem)
        pltpu.async_copy(ngv, neg_hbm.at[g, pl.ds(off, CHM)], osem)

    issue(0, 0)

    def pair(c2, _):
        c = 2 * c2
        wait(0)
        issue(c + 1, 1)

        @pl.when(c2 > 0)
        def _wait_out0():
            wait_out(0)

        compute(c, 0)
        wait(1)

        @pl.when(c2 < NCH2 // 2 - 1)
        def _issue_next():
            issue(c + 2, 0)

        @pl.when(c2 > 0)
        def _wait_out1():
            wait_out(1)

        compute(c + 1, 1)
        return 0

    lax.fori_loop(0, NCH2 // 2, pair, 0)
    wait_out(0)
    wait_out(1)


BE = 12800
GE = E // BE


def _red_body(pp_ref, np_ref, pos_ref, neg_ref):
    pos_ref[...] = jnp.sum(pp_ref[...], axis=0, keepdims=True)
    neg_ref[...] = jnp.sum(np_ref[...], axis=0, keepdims=True)


def _red(posp, negp):
    return pl.pallas_call(
        _red_body,
        grid=(GE,),
        in_specs=[
            pl.BlockSpec((NW // 2, BE), lambda i: (0, i)),
            pl.BlockSpec((NW // 2, BE), lambda i: (0, i)),
        ],
        out_specs=[
            pl.BlockSpec((1, BE), lambda i: (0, i)),
            pl.BlockSpec((1, BE), lambda i: (0, i)),
        ],
        out_shape=[
            jax.ShapeDtypeStruct((1, E), jnp.float32),
            jax.ShapeDtypeStruct((1, E), jnp.float32),
        ],
    )(posp, negp)


@functools.lru_cache(maxsize=None)
def _deg_kernel():
    return pl.kernel(
        _deg_body,
        out_type=[
            jax.ShapeDtypeStruct((NW, N), jnp.float32),
            jax.ShapeDtypeStruct((E,), jnp.int32),
        ],
        mesh=_mesh(),
        compiler_params=_SC_PARAMS,
        scratch_types=[
            pltpu.VMEM((N,), jnp.float32),
            pltpu.VMEM((CHD,), jnp.int32),
            pltpu.VMEM((CHD,), jnp.float32),
            pltpu.VMEM((CHD,), jnp.int32),
            pltpu.VMEM((CHD,), jnp.int32),
            pltpu.SemaphoreType.DMA,
        ],
    )


@functools.lru_cache(maxsize=None)
def _main_kernel():
    return pl.kernel(
        _main_body,
        out_type=jax.ShapeDtypeStruct((D, N), jnp.float32),
        mesh=_mesh(),
        compiler_params=_SC_PARAMS,
        scratch_types=[
            pltpu.VMEM((N,), jnp.float32),
            pltpu.VMEM((2, N), jnp.int32),
            pltpu.VMEM((CPT, N), jnp.float32),
            pltpu.VMEM((CHM,), jnp.int32),
            pltpu.VMEM((CHM,), jnp.int32),
            pltpu.VMEM((CHM,), jnp.int32),
            pltpu.VMEM((CHM,), jnp.int32),
            pltpu.VMEM((CHM,), jnp.float32),
            pltpu.VMEM((CHM,), jnp.float32),
            pltpu.SemaphoreType.DMA,
            pltpu.SemaphoreType.DMA,
        ],
    )


@functools.lru_cache(maxsize=None)
def _dec_kernel():
    return pl.kernel(
        _dec_body,
        out_type=[
            jax.ShapeDtypeStruct((NW // 2, E), jnp.float32),
            jax.ShapeDtypeStruct((NW // 2, E), jnp.float32),
        ],
        mesh=_mesh(),
        compiler_params=_SC_PARAMS,
        scratch_types=[
            pltpu.VMEM((4, N), jnp.int32),
            pltpu.VMEM((CHM,), jnp.int32),
            pltpu.VMEM((CHM,), jnp.int32),
            pltpu.VMEM((CHM,), jnp.int32),
            pltpu.VMEM((CHM,), jnp.int32),
            pltpu.VMEM((CHM,), jnp.int32),
            pltpu.VMEM((CHM,), jnp.int32),
            pltpu.VMEM((CHM,), jnp.float32),
            pltpu.VMEM((CHM,), jnp.float32),
            pltpu.VMEM((CHM,), jnp.float32),
            pltpu.VMEM((CHM,), jnp.float32),
            pltpu.SemaphoreType.DMA,
            pltpu.SemaphoreType.DMA,
            pltpu.SemaphoreType.DMA,
            pltpu.SemaphoreType.DMA,
        ],
    )


def kernel(dynamic_node_feats, node_ids, edge_index, edge_feats, perm,
           proj_W, proj_b, node_emb, conv_z_W, conv_z_b, conv_r_W, conv_r_b,
           conv_h_W, conv_h_b, lin_z_W, lin_z_b, lin_r_W, lin_r_b,
           lin_h_W, lin_h_b):
    ei = edge_index
    ew = edge_feats[:, 0]
    perm32 = perm.astype(jnp.int32)

    degp, ndst = _deg_kernel()(ei.reshape(2 * E), ew, perm32)
    ht, hp, dinv = _prep(dynamic_node_feats, proj_W, proj_b.reshape(1, D),
                         node_emb, degp)
    q = _main_kernel()(hp, dinv, ei, ew)
    z, zp = _zk(q, dinv, ht,
                conv_z_W, conv_z_b.reshape(1, D), lin_z_W[:D], lin_z_b.reshape(1, D),
                conv_h_W, conv_h_b.reshape(1, D), lin_h_W[:D], lin_h_b.reshape(1, D))
    posp, negp = _dec_kernel()(zp, ei, ndst)
    pos, neg = _red(posp, negp)
    return pos.reshape(E), neg.reshape(E), z


# speedup vs baseline: 1.3611x; 1.3611x over previous
import functools

import jax
import jax.numpy as jnp
from jax import lax
from jax.experimental import pallas as pl
from jax.experimental.pallas import tpu as pltpu
from jax.experimental.pallas import tpu_sc as plsc

N = 10000
E = 320000
D = 128
DIN = 128

NC = 2
NS = 16
NW = NC * NS
CPT = D // NW
L = 16

def _mesh():
    return plsc.VectorSubcoreMesh(core_axis_name="c", subcore_axis_name="s",
                                  num_cores=NC, num_subcores=NS)


_SC_PARAMS = pltpu.CompilerParams(needs_layout_passes=False,
                                  use_tc_tiling_on_sc=False)

CHD = 2000
CHM = 3200
NCHM = E // CHM
EPW = E // NW


def _wid():
    return lax.axis_index("s") * NC + lax.axis_index("c")


def _deg_body(eif_hbm, ew_hbm, perm_hbm, degp_hbm, ndst_hbm,
              deg_v, di_v, ew_v, pm_v, nd_v, sem):
    w = _wid()
    zero = jnp.zeros((L,), jnp.float32)

    @plsc.parallel_loop(0, N // L, 1, unroll=8)
    def zbody(i):
        deg_v[pl.ds(i * L, L)] = zero

    base = w * EPW

    def cbody(c, _):
        off = base + c * CHD
        pltpu.sync_copy(eif_hbm.at[pl.ds(E + off, CHD)], di_v)
        pltpu.sync_copy(ew_hbm.at[pl.ds(off, CHD)], ew_v)
        pltpu.sync_copy(perm_hbm.at[pl.ds(off, CHD)], pm_v)
        eoff = jnp.full((L,), E, jnp.int32)

        @plsc.parallel_loop(0, CHD // L, 1, unroll=8)
        def pbody(i):
            sl = pl.ds(i * L, L)
            pm_v[sl] = pm_v[sl] + eoff

        cp = pltpu.async_copy(eif_hbm.at[pm_v], nd_v, sem)

        @plsc.parallel_loop(0, CHD // L, 1, unroll=8)
        def gbody(g):
            idx = di_v[pl.ds(g * L, L)]
            val = ew_v[pl.ds(g * L, L)]
            plsc.addupdate_scatter(deg_v, [idx], val)

        cp.wait()
        pltpu.sync_copy(nd_v, ndst_hbm.at[pl.ds(off, CHD)])
        return 0

    lax.fori_loop(0, EPW // CHD, cbody, 0)
    pltpu.sync_copy(deg_v, degp_hbm.at[w])


BR = 1024
GR = (N + BR - 1) // BR


def _pack_cols(x):
    lo = lax.bitcast_convert_type(x[:, :D // 2].astype(jnp.bfloat16),
                                  jnp.uint16).astype(jnp.uint32)
    hi = lax.bitcast_convert_type(x[:, D // 2:].astype(jnp.bfloat16),
                                  jnp.uint16).astype(jnp.uint32)
    return lax.bitcast_convert_type(lo | (hi << 16), jnp.int32)


def _prep_body(x_ref, pw_ref, pb_ref, emb_ref, degp_ref, ht_ref, hp_ref, dinv_ref):
    hb = (
        jnp.dot(x_ref[...], pw_ref[...], preferred_element_type=jnp.float32)
        + pb_ref[...]
        + emb_ref[...]
    )
    ht_ref[...] = hb.T
    deg = jnp.sum(degp_ref[...], axis=0, keepdims=True) + 1.0
    dv = jnp.where(deg > 0, lax.rsqrt(deg), 0.0)
    dinv_ref[...] = dv
    hp_ref[...] = _pack_cols(hb * dv.T).T


def _prep(x, pw, pb, emb, degp):
    return pl.pallas_call(
        _prep_body,
        grid=(GR,),
        in_specs=[
            pl.BlockSpec((BR, DIN), lambda i: (i, 0)),
            pl.BlockSpec((DIN, D), lambda i: (0, 0)),
            pl.BlockSpec((1, D), lambda i: (0, 0)),
            pl.BlockSpec((BR, D), lambda i: (i, 0)),
            pl.BlockSpec((NW, BR), lambda i: (0, i)),
        ],
        out_specs=[
            pl.BlockSpec((D, BR), lambda i: (0, i)),
            pl.BlockSpec((D // 2, BR), lambda i: (0, i)),
            pl.BlockSpec((1, BR), lambda i: (0, i)),
        ],
        out_shape=[
            jax.ShapeDtypeStruct((D, N), jnp.float32),
            jax.ShapeDtypeStruct((D // 2, N), jnp.int32),
            jax.ShapeDtypeStruct((1, N), jnp.float32),
        ],
    )(x, pw, pb, emb, degp)


def _main_body(hp_hbm, dinv_hbm, ei_hbm, ew_hbm, q_hbm,
               dinv_v, h_v, q_v,
               src0, src1, dst0, dst1, ew0, ew1, se0, se1):
    w = _wid()
    pltpu.sync_copy(dinv_hbm.at[0], dinv_v)
    pltpu.sync_copy(hp_hbm.at[pl.ds(2 * w, 2)], h_v)

    zero = jnp.zeros((L,), jnp.float32)
    for r in range(CPT):
        @plsc.parallel_loop(0, N // L, 1, unroll=8)
        def zbody(i, r=r):
            q_v[r, pl.ds(i * L, L)] = zero

    bufs = ((src0, dst0, ew0, se0), (src1, dst1, ew1, se1))

    def issue(c, b):
        sv, dv, wv, sem = bufs[b]
        off = c * CHM
        pltpu.async_copy(ei_hbm.at[0, pl.ds(off, CHM)], sv, sem)
        pltpu.async_copy(ei_hbm.at[1, pl.ds(off, CHM)], dv, sem)
        pltpu.async_copy(ew_hbm.at[pl.ds(off, CHM)], wv, sem)

    def wait(b):
        sv, dv, wv, sem = bufs[b]
        pltpu.make_async_copy(ei_hbm.at[0, pl.ds(0, CHM)], sv, sem).wait()
        pltpu.make_async_copy(ei_hbm.at[1, pl.ds(0, CHM)], dv, sem).wait()
        pltpu.make_async_copy(ew_hbm.at[pl.ds(0, CHM)], wv, sem).wait()

    def compute(b):
        sv, dv, wv, _ = bufs[b]

        @plsc.parallel_loop(0, CHM // L, 1, unroll=8)
        def gbody(g):
            s16 = sv[pl.ds(g * L, L)]
            d16 = dv[pl.ds(g * L, L)]
            w16 = wv[pl.ds(g * L, L)]
            for r2 in range(2):
                rf = jnp.full((L,), r2, jnp.int32)
                pg = plsc.load_gather(h_v, [rf, s16])
                lo, hi = plsc.unpack(plsc.bitcast(pg, jnp.bfloat16),
                                     format=plsc.PackFormat.INTERLEAVED)
                plsc.addupdate_scatter(q_v, [rf, d16], lo * w16)
                rf2 = jnp.full((L,), 2 + r2, jnp.int32)
                plsc.addupdate_scatter(q_v, [rf2, d16], hi * w16)

    issue(0, 0)

    def pair(c2, _):
        c = 2 * c2
        wait(0)
        issue(c + 1, 1)
        compute(0)
        wait(1)

        @pl.when(c2 < NCHM // 2 - 1)
        def _issue_next():
            issue(c + 2, 0)

        compute(1)
        return 0

    lax.fori_loop(0, NCHM // 2, pair, 0)

    for r in range(CPT):
        @plsc.parallel_loop(0, N // L, 1, unroll=4)
        def qbody(i, r=r):
            sl = pl.ds(i * L, L)
            q_v[r, sl] = q_v[r, sl] * dinv_v[sl]

    pltpu.sync_copy(q_v.at[pl.ds(0, 2)], q_hbm.at[pl.ds(2 * w, 2)])
    pltpu.sync_copy(q_v.at[pl.ds(2, 2)], q_hbm.at[pl.ds(D // 2 + 2 * w, 2)])


def _z_body(q_ref, dinv_ref, ht_ref, czw_ref, czb_ref, lzw_ref, lzb_ref,
            chw_ref, chb_ref, lhw_ref, lhb_ref, z_ref, zp_ref):
    wz = jnp.dot(czw_ref[...], lzw_ref[...], preferred_element_type=jnp.float32)
    bz = jnp.dot(czb_ref[...], lzw_ref[...], preferred_element_type=jnp.float32) + lzb_ref[...]
    wh = jnp.dot(chw_ref[...], lhw_ref[...], preferred_element_type=jnp.float32)
    bh = jnp.dot(chb_ref[...], lhw_ref[...], preferred_element_type=jnp.float32) + lhb_ref[...]
    dv = dinv_ref[...]
    pt = q_ref[...] + dv * dv * ht_ref[...]
    mz = lax.dot_general(pt, wz, (((0,), (0,)), ((), ())),
                         preferred_element_type=jnp.float32)
    mh = lax.dot_general(pt, wh, (((0,), (0,)), ((), ())),
                         preferred_element_type=jnp.float32)
    zb = (1.0 - jax.nn.sigmoid(mz + bz)) * jnp.tanh(mh + bh)
    z_ref[...] = zb
    zp_ref[...] = _pack_cols(zb).T


def _zk(q, dinv, ht, czw, czb, lzw, lzb, chw, chb, lhw, lhb):
    full = lambda i: (0, 0)
    return pl.pallas_call(
        _z_body,
        grid=(GR,),
        in_specs=[
            pl.BlockSpec((D, BR), lambda i: (0, i)),
            pl.BlockSpec((1, BR), lambda i: (0, i)),
            pl.BlockSpec((D, BR), lambda i: (0, i)),
            pl.BlockSpec((D, D), full),
            pl.BlockSpec((1, D), full),
            pl.BlockSpec((D, D), full),
            pl.BlockSpec((1, D), full),
            pl.BlockSpec((D, D), full),
            pl.BlockSpec((1, D), full),
            pl.BlockSpec((D, D), full),
            pl.BlockSpec((1, D), full),
        ],
        out_specs=[
            pl.BlockSpec((BR, D), lambda i: (i, 0)),
            pl.BlockSpec((D // 2, BR), lambda i: (0, i)),
        ],
        out_shape=[
            jax.ShapeDtypeStruct((N, D), jnp.float32),
            jax.ShapeDtypeStruct((D // 2, N), jnp.int32),
        ],
    )(q, dinv, ht, czw, czb, lzw, lzb, chw, chb, lhw, lhb)


def _dec_body(zt_hbm, ei_hbm, ndst_hbm, pos_hbm, neg_hbm,
              z_v, src0, src1, dst0, dst1, nd0, nd1,
              pos0, pos1, neg0, neg1, se0, se1, so0, so1):
    w = _wid()
    pltpu.sync_copy(zt_hbm.at[pl.ds(2 * w, 2)], z_v)
    ebufs = ((src0, dst0, nd0, se0), (src1, dst1, nd1, se1))
    obufs = ((pos0, neg0, so0), (pos1, neg1, so1))

    def issue(c, b):
        sv, dv, nv, sem = ebufs[b]
        off = c * CHM
        pltpu.async_copy(ei_hbm.at[0, pl.ds(off, CHM)], sv, sem)
        pltpu.async_copy(ei_hbm.at[1, pl.ds(off, CHM)], dv, sem)
        pltpu.async_copy(ndst_hbm.at[pl.ds(off, CHM)], nv, sem)

    def wait(b):
        sv, dv, nv, sem = ebufs[b]
        pltpu.make_async_copy(ei_hbm.at[0, pl.ds(0, CHM)], sv, sem).wait()
        pltpu.make_async_copy(ei_hbm.at[1, pl.ds(0, CHM)], dv, sem).wait()
        pltpu.make_async_copy(ndst_hbm.at[pl.ds(0, CHM)], nv, sem).wait()

    def wait_out(b):
        pv, ngv, sem = obufs[b]
        pltpu.make_async_copy(pv, pos_hbm.at[w, pl.ds(0, CHM)], sem).wait()
        pltpu.make_async_copy(ngv, neg_hbm.at[w, pl.ds(0, CHM)], sem).wait()

    def compute(c, b):
        sv, dv, nv, _ = ebufs[b]
        pv, ngv, osem = obufs[b]

        @plsc.parallel_loop(0, CHM // L, 1, unroll=8)
        def gbody(g):
            s16 = sv[pl.ds(g * L, L)]
            d16 = dv[pl.ds(g * L, L)]
            n16 = nv[pl.ds(g * L, L)]
            pacc = jnp.zeros((L,), jnp.float32)
            nacc = jnp.zeros((L,), jnp.float32)
            for r2 in range(2):
                rf = jnp.full((L,), r2, jnp.int32)
                fmt = plsc.PackFormat.INTERLEAVED
                zsl, zsh = plsc.unpack(
                    plsc.bitcast(plsc.load_gather(z_v, [rf, s16]),
                                 jnp.bfloat16), format=fmt)
                zdl, zdh = plsc.unpack(
                    plsc.bitcast(plsc.load_gather(z_v, [rf, d16]),
                                 jnp.bfloat16), format=fmt)
                znl, znh = plsc.unpack(
                    plsc.bitcast(plsc.load_gather(z_v, [rf, n16]),
                                 jnp.bfloat16), format=fmt)
                pacc = pacc + zsl * zdl + zsh * zdh
                nacc = nacc + zsl * znl + zsh * znh
            pv[pl.ds(g * L, L)] = pacc
            ngv[pl.ds(g * L, L)] = nacc

        off = c * CHM
        pltpu.async_copy(pv, pos_hbm.at[w, pl.ds(off, CHM)], osem)
        pltpu.async_copy(ngv, neg_hbm.at[w, pl.ds(off, CHM)], osem)

    issue(0, 0)

    def pair(c2, _):
        c = 2 * c2
        wait(0)
        issue(c + 1, 1)

        @pl.when(c2 > 0)
        def _wait_out0():
            wait_out(0)

        compute(c, 0)
        wait(1)

        @pl.when(c2 < NCHM // 2 - 1)
        def _issue_next():
            issue(c + 2, 0)

        @pl.when(c2 > 0)
        def _wait_out1():
            wait_out(1)

        compute(c + 1, 1)
        return 0

    lax.fori_loop(0, NCHM // 2, pair, 0)
    wait_out(0)
    wait_out(1)


BE = 12800
GE = E // BE


def _red_body(pp_ref, np_ref, pos_ref, neg_ref):
    pos_ref[...] = jnp.sum(pp_ref[...], axis=0, keepdims=True)
    neg_ref[...] = jnp.sum(np_ref[...], axis=0, keepdims=True)


def _red(posp, negp):
    return pl.pallas_call(
        _red_body,
        grid=(GE,),
        in_specs=[
            pl.BlockSpec((NW, BE), lambda i: (0, i)),
            pl.BlockSpec((NW, BE), lambda i: (0, i)),
        ],
        out_specs=[
            pl.BlockSpec((1, BE), lambda i: (0, i)),
            pl.BlockSpec((1, BE), lambda i: (0, i)),
        ],
        out_shape=[
            jax.ShapeDtypeStruct((1, E), jnp.float32),
            jax.ShapeDtypeStruct((1, E), jnp.float32),
        ],
    )(posp, negp)


@functools.lru_cache(maxsize=None)
def _deg_kernel():
    return pl.kernel(
        _deg_body,
        out_type=[
            jax.ShapeDtypeStruct((NW, N), jnp.float32),
            jax.ShapeDtypeStruct((E,), jnp.int32),
        ],
        mesh=_mesh(),
        compiler_params=_SC_PARAMS,
        scratch_types=[
            pltpu.VMEM((N,), jnp.float32),
            pltpu.VMEM((CHD,), jnp.int32),
            pltpu.VMEM((CHD,), jnp.float32),
            pltpu.VMEM((CHD,), jnp.int32),
            pltpu.VMEM((CHD,), jnp.int32),
            pltpu.SemaphoreType.DMA,
        ],
    )


@functools.lru_cache(maxsize=None)
def _main_kernel():
    return pl.kernel(
        _main_body,
        out_type=jax.ShapeDtypeStruct((D, N), jnp.float32),
        mesh=_mesh(),
        compiler_params=_SC_PARAMS,
        scratch_types=[
            pltpu.VMEM((N,), jnp.float32),
            pltpu.VMEM((2, N), jnp.int32),
            pltpu.VMEM((CPT, N), jnp.float32),
            pltpu.VMEM((CHM,), jnp.int32),
            pltpu.VMEM((CHM,), jnp.int32),
            pltpu.VMEM((CHM,), jnp.int32),
            pltpu.VMEM((CHM,), jnp.int32),
            pltpu.VMEM((CHM,), jnp.float32),
            pltpu.VMEM((CHM,), jnp.float32),
            pltpu.SemaphoreType.DMA,
            pltpu.SemaphoreType.DMA,
        ],
    )


@functools.lru_cache(maxsize=None)
def _dec_kernel():
    return pl.kernel(
        _dec_body,
        out_type=[
            jax.ShapeDtypeStruct((NW, E), jnp.float32),
            jax.ShapeDtypeStruct((NW, E), jnp.float32),
        ],
        mesh=_mesh(),
        compiler_params=_SC_PARAMS,
        scratch_types=[
            pltpu.VMEM((2, N), jnp.int32),
            pltpu.VMEM((CHM,), jnp.int32),
            pltpu.VMEM((CHM,), jnp.int32),
            pltpu.VMEM((CHM,), jnp.int32),
            pltpu.VMEM((CHM,), jnp.int32),
            pltpu.VMEM((CHM,), jnp.int32),
            pltpu.VMEM((CHM,), jnp.int32),
            pltpu.VMEM((CHM,), jnp.float32),
            pltpu.VMEM((CHM,), jnp.float32),
            pltpu.VMEM((CHM,), jnp.float32),
            pltpu.VMEM((CHM,), jnp.float32),
            pltpu.SemaphoreType.DMA,
            pltpu.SemaphoreType.DMA,
            pltpu.SemaphoreType.DMA,
            pltpu.SemaphoreType.DMA,
        ],
    )


def kernel(dynamic_node_feats, node_ids, edge_index, edge_feats, perm,
           proj_W, proj_b, node_emb, conv_z_W, conv_z_b, conv_r_W, conv_r_b,
           conv_h_W, conv_h_b, lin_z_W, lin_z_b, lin_r_W, lin_r_b,
           lin_h_W, lin_h_b):
    ei = edge_index
    ew = edge_feats[:, 0]
    perm32 = perm.astype(jnp.int32)

    degp, ndst = _deg_kernel()(ei.reshape(2 * E), ew, perm32)
    ht, hp, dinv = _prep(dynamic_node_feats, proj_W, proj_b.reshape(1, D),
                         node_emb, degp)
    q = _main_kernel()(hp, dinv, ei, ew)
    z, zp = _zk(q, dinv, ht,
                conv_z_W, conv_z_b.reshape(1, D), lin_z_W[:D], lin_z_b.reshape(1, D),
                conv_h_W, conv_h_b.reshape(1, D), lin_h_W[:D], lin_h_b.reshape(1, D))
    posp, negp = _dec_kernel()(zp, ei, ndst)
    pos, neg = _red(posp, negp)
    return pos.reshape(E), neg.reshape(E), z
